# baseline (speedup 1.0000x reference)
import jax
import jax.numpy as jnp
from jax import lax
from jax.experimental import pallas as pl
from jax.experimental.pallas import tpu as pltpu

_B = 1024
_NI = 512
_NO = 64
_H = 128

_OB = 8
_G = _NO // _OB


def _fused_body(l1_ref, l2_ref, l3_ref, state_ref, opt_ref, out_ref, v_s):
    o = pl.program_id(0)

    @pl.when(o < _G)
    def _build_v():
        l1b = l1_ref[...]
        l2b = l2_ref[...]
        l3b = l3_ref[...]
        u = lax.dot_general(l3b, l2b, (((1,), (2,)), ((0,), (0,))),
                            preferred_element_type=jnp.float32)
        vrow = lax.dot_general(u, l1b, (((2,), (2,)), ((0,), (0,))),
                               preferred_element_type=jnp.float32)
        v_s[pl.ds(o * _OB, _OB), :] = vrow.reshape(_OB, _NI)

    @pl.when(o == _G)
    def _select():
        scores = lax.dot_general(state_ref[...], v_s[...],
                                 (((1,), (1,)), ((), ())),
                                 preferred_element_type=jnp.float32)
        onehot = (opt_ref[...] == lax.broadcasted_iota(jnp.int32, (1, _NO), 1))
        out_ref[...] = jnp.sum(jnp.where(onehot, scores, 0.0), axis=1,
                               keepdims=True)


def kernel(state, option, action, linear1, linear2, linear3):
    opt = option.astype(jnp.int32).reshape(_B, 1)
    clamp = lambda o: (jnp.minimum(o, _G - 1), 0, 0)
    out = pl.pallas_call(
        _fused_body,
        grid=(_G + 1,),
        in_specs=[
            pl.BlockSpec((_OB, _NI, _H), clamp),
            pl.BlockSpec((_OB, _H, _H), clamp),
            pl.BlockSpec((_OB, _H, 1), clamp),
            pl.BlockSpec((_B, _NI), lambda o: (0, 0)),
            pl.BlockSpec((_B, 1), lambda o: (0, 0)),
        ],
        out_specs=pl.BlockSpec((_B, 1), lambda o: (0, 0)),
        out_shape=jax.ShapeDtypeStruct((_B, 1), jnp.float32),
        scratch_shapes=[pltpu.VMEM((_NO, _NI), jnp.float32)],
    )(linear1, linear2, linear3, state, opt)
    return out


# speedup vs baseline: 1.0758x; 1.0758x over previous
import jax
import jax.numpy as jnp
from jax import lax
from jax.experimental import pallas as pl
from jax.experimental.pallas import tpu as pltpu

_B = 1024
_NI = 512
_NO = 64
_H = 128

_OB = 32
_G = _NO // _OB


def _fused_body(l1_ref, l2_ref, l3_ref, state_ref, opt_ref, out_ref, v_s):
    o = pl.program_id(0)

    @pl.when(o < _G)
    def _build_v():
        l1b = l1_ref[...]
        l2b = l2_ref[...]
        l3b = l3_ref[...]
        u = lax.dot_general(l3b, l2b, (((1,), (2,)), ((0,), (0,))),
                            preferred_element_type=jnp.float32)
        vrow = lax.dot_general(u, l1b, (((2,), (2,)), ((0,), (0,))),
                               preferred_element_type=jnp.float32)
        v_s[pl.ds(o * _OB, _OB), :] = vrow.reshape(_OB, _NI)

    @pl.when(o == _G)
    def _select():
        scores = lax.dot_general(state_ref[...], v_s[...],
                                 (((1,), (1,)), ((), ())),
                                 preferred_element_type=jnp.float32)
        onehot = (opt_ref[...] == lax.broadcasted_iota(jnp.int32, (1, _NO), 1))
        out_ref[...] = jnp.sum(jnp.where(onehot, scores, 0.0), axis=1,
                               keepdims=True)


def kernel(state, option, action, linear1, linear2, linear3):
    opt = option.astype(jnp.int32).reshape(_B, 1)
    clamp = lambda o: (jnp.minimum(o, _G - 1), 0, 0)
    out = pl.pallas_call(
        _fused_body,
        grid=(_G + 1,),
        in_specs=[
            pl.BlockSpec((_OB, _NI, _H), clamp),
            pl.BlockSpec((_OB, _H, _H), clamp),
            pl.BlockSpec((_OB, _H, 1), clamp),
            pl.BlockSpec((_B, _NI), lambda o: (0, 0)),
            pl.BlockSpec((_B, 1), lambda o: (0, 0)),
        ],
        out_specs=pl.BlockSpec((_B, 1), lambda o: (0, 0)),
        out_shape=jax.ShapeDtypeStruct((_B, 1), jnp.float32),
        scratch_shapes=[pltpu.VMEM((_NO, _NI), jnp.float32)],
    )(linear1, linear2, linear3, state, opt)
    return out


# speedup vs baseline: 1.0901x; 1.0133x over previous
import jax
import jax.numpy as jnp
from jax import lax
from jax.experimental import pallas as pl
from jax.experimental.pallas import tpu as pltpu

_B = 1024
_NI = 512
_NO = 64
_H = 128

_OB = 16
_G = _NO // _OB


def _fused_body(l1_ref, l2_ref, l3_ref, state_ref, opt_ref, out_ref, v_s):
    o = pl.program_id(0)

    @pl.when(o < _G)
    def _build_v():
        l1b = l1_ref[...]
        l2b = l2_ref[...]
        l3b = l3_ref[...]
        u = lax.dot_general(l3b, l2b, (((1,), (2,)), ((0,), (0,))),
                            preferred_element_type=jnp.float32)
        vrow = lax.dot_general(u, l1b, (((2,), (2,)), ((0,), (0,))),
                               preferred_element_type=jnp.float32)
        v_s[pl.ds(o * _OB, _OB), :] = vrow.reshape(_OB, _NI)

    @pl.when(o == _G)
    def _select():
        scores = lax.dot_general(state_ref[...], v_s[...],
                                 (((1,), (1,)), ((), ())),
                                 preferred_element_type=jnp.float32)
        onehot = (opt_ref[...] == lax.broadcasted_iota(jnp.int32, (1, _NO), 1))
        out_ref[...] = jnp.sum(jnp.where(onehot, scores, 0.0), axis=1,
                               keepdims=True)


def kernel(state, option, action, linear1, linear2, linear3):
    opt = option.astype(jnp.int32).reshape(_B, 1)
    clamp = lambda o: (jnp.minimum(o, _G - 1), 0, 0)
    out = pl.pallas_call(
        _fused_body,
        grid=(_G + 1,),
        in_specs=[
            pl.BlockSpec((_OB, _NI, _H), clamp),
            pl.BlockSpec((_OB, _H, _H), clamp),
            pl.BlockSpec((_OB, _H, 1), clamp),
            pl.BlockSpec((_B, _NI), lambda o: (0, 0)),
            pl.BlockSpec((_B, 1), lambda o: (0, 0)),
        ],
        out_specs=pl.BlockSpec((_B, 1), lambda o: (0, 0)),
        out_shape=jax.ShapeDtypeStruct((_B, 1), jnp.float32),
        scratch_shapes=[pltpu.VMEM((_NO, _NI), jnp.float32)],
    )(linear1, linear2, linear3, state, opt)
    return out
